# Initial kernel scaffold; baseline (speedup 1.0000x reference)
import jax
import jax.numpy as jnp
from jax.experimental import pallas as pl


def kernel(item_feat, user_feat, edge_index_i2u, edge_index_u2i, W_i2u, b_i2u, W_u2i, b_u2i):
    raise NotImplementedError("write your pallas kernel here")


# speedup vs baseline: 8.9808x; 8.9808x over previous
import functools

import jax
import jax.numpy as jnp
from jax import lax
from jax.experimental import pallas as pl
from jax.experimental.pallas import tpu as pltpu
from jax.experimental.pallas import tpu_sc as plsc

N = 10000
D = 128
E = 320000
NTILES = 16
NP = 10240
ROWS_PT = NP // NTILES
CHUNK = 128
CPT = 160
GRP = 40
EP = NTILES * CPT * CHUNK
NROWS_E = EP // CHUNK

_sc_mesh = plsc.VectorSubcoreMesh(core_axis_name="c", subcore_axis_name="s")


@functools.partial(
    pl.kernel,
    out_type=jax.ShapeDtypeStruct((4, 1, NP), jnp.float32),
    mesh=_sc_mesh,
    scratch_types=[
        pltpu.VMEM_SHARED((NP,), jnp.float32),
        pltpu.VMEM_SHARED((NP,), jnp.float32),
        pltpu.VMEM((CPT, CHUNK), jnp.int32),
        pltpu.VMEM((CHUNK,), jnp.float32),
        pltpu.VMEM((640,), jnp.float32),
    ],
)
def _sc_degrees(idx_hbm, deg_hbm, hist_src, hist_dst, idx_v, ones_v, zbuf_v):
    c = lax.axis_index("c")
    s = lax.axis_index("s")
    row0 = s * ROWS_PT

    for j in range(CHUNK // 16):
        ones_v[pl.ds(j * 16, 16)] = jnp.ones((16,), jnp.float32)

    def fill_zeros(i, carry):
        zbuf_v[pl.ds(i * 16, 16)] = jnp.zeros((16,), jnp.float32)
        return carry

    lax.fori_loop(0, 640 // 16, fill_zeros, 0)
    pltpu.sync_copy(zbuf_v.at[pl.ds(0, ROWS_PT)], hist_src.at[pl.ds(row0, ROWS_PT)])
    pltpu.sync_copy(zbuf_v.at[pl.ds(0, ROWS_PT)], hist_dst.at[pl.ds(row0, ROWS_PT)])
    plsc.subcore_barrier()

    pltpu.sync_copy(idx_hbm.at[c, 0, pl.ds(s * CPT, CPT)], idx_v)

    def scat_src(ch, carry):
        pltpu.sync_copy(ones_v, hist_src.at[idx_v.at[ch]], add=True)
        return carry

    lax.fori_loop(0, CPT, scat_src, 0)

    pltpu.sync_copy(idx_hbm.at[c, 1, pl.ds(s * CPT, CPT)], idx_v)

    def scat_dst(ch, carry):
        pltpu.sync_copy(ones_v, hist_dst.at[idx_v.at[ch]], add=True)
        return carry

    lax.fori_loop(0, CPT, scat_dst, 0)
    plsc.subcore_barrier()

    pltpu.sync_copy(hist_src.at[pl.ds(row0, ROWS_PT)],
                    deg_hbm.at[2 * c, 0, pl.ds(row0, ROWS_PT)])
    pltpu.sync_copy(hist_dst.at[pl.ds(row0, ROWS_PT)],
                    deg_hbm.at[2 * c + 1, 0, pl.ds(row0, ROWS_PT)])


@functools.partial(
    pl.kernel,
    out_type=jax.ShapeDtypeStruct((2, NP, D), jnp.float32),
    mesh=_sc_mesh,
    scratch_types=[
        pltpu.VMEM_SHARED((NP, D), jnp.float32),
        pltpu.VMEM((GRP, CHUNK), jnp.int32),
        pltpu.VMEM((GRP, CHUNK), jnp.int32),
        pltpu.VMEM((CHUNK, D), jnp.float32),
        pltpu.SemaphoreType.DMA,
    ],
)
def _sc_aggregate(h_hbm, idx_hbm, agg_hbm, agg_sh, sidx_v, didx_v, stage_v, sem):
    c = lax.axis_index("c")
    s = lax.axis_index("s")
    row0 = s * ROWS_PT

    def zero_row(i, carry):
        for j in range(D // 16):
            stage_v[i, pl.ds(j * 16, 16)] = jnp.zeros((16,), jnp.float32)
        return carry

    lax.fori_loop(0, CHUNK, zero_row, 0)
    for k in range(ROWS_PT // CHUNK):
        pltpu.sync_copy(stage_v, agg_sh.at[pl.ds(row0 + k * CHUNK, CHUNK)])
    plsc.subcore_barrier()

    for g in range(CPT // GRP):
        base = s * CPT + g * GRP
        pltpu.sync_copy(idx_hbm.at[c, 0, pl.ds(base, GRP)], sidx_v)
        pltpu.sync_copy(idx_hbm.at[c, 1, pl.ds(base, GRP)], didx_v)

        def body(ch, carry):
            pltpu.async_copy(h_hbm.at[c].at[sidx_v.at[ch]], stage_v, sem).wait()
            pltpu.sync_copy(stage_v, agg_sh.at[didx_v.at[ch]], add=True)
            return carry

        lax.fori_loop(0, GRP, body, 0)

    plsc.subcore_barrier()
    pltpu.sync_copy(agg_sh.at[pl.ds(row0, ROWS_PT)],
                    agg_hbm.at[c, pl.ds(row0, ROWS_PT)])


_PBLK = NP // 8


def _tc_prep_body(feat_ref, deg_ref, h_ref):
    d = deg_ref[0]
    h_ref[0] = feat_ref[0] * lax.rsqrt(jnp.maximum(d, 1.0))


def _tc_prep(feats, degs_src):
    return pl.pallas_call(
        _tc_prep_body,
        grid=(2, NP // _PBLK),
        in_specs=[
            pl.BlockSpec((1, _PBLK, D), lambda r, i: (r, i, 0)),
            pl.BlockSpec((1, _PBLK, 1), lambda r, i: (r, i, 0)),
        ],
        out_specs=pl.BlockSpec((1, _PBLK, D), lambda r, i: (r, i, 0)),
        out_shape=jax.ShapeDtypeStruct((2, NP, D), jnp.float32),
    )(feats, degs_src)


_FBLK = 1000


def _tc_final_body(agg_ref, deg_ref, w_ref, b_ref, out_ref):
    a = agg_ref[0] * lax.rsqrt(jnp.maximum(deg_ref[0], 1.0))
    out_ref[0] = jnp.dot(a, w_ref[0], preferred_element_type=jnp.float32) + b_ref[0]


def _tc_final(agg, degs_dst, W, b):
    return pl.pallas_call(
        _tc_final_body,
        grid=(2, N // _FBLK),
        in_specs=[
            pl.BlockSpec((1, _FBLK, D), lambda r, i: (r, i, 0)),
            pl.BlockSpec((1, _FBLK, 1), lambda r, i: (r, i, 0)),
            pl.BlockSpec((1, D, D), lambda r, i: (r, 0, 0)),
            pl.BlockSpec((1, 1, D), lambda r, i: (r, 0, 0)),
        ],
        out_specs=pl.BlockSpec((1, _FBLK, D), lambda r, i: (r, i, 0)),
        out_shape=jax.ShapeDtypeStruct((2, N, D), jnp.float32),
    )(agg, degs_dst, W, b)


def kernel(item_feat, user_feat, edge_index_i2u, edge_index_u2i,
           W_i2u, b_i2u, W_u2i, b_u2i):
    pad = (jnp.arange(EP - E, dtype=jnp.int32) % (NP - N)) + N

    def prep_idx(e):
        src = jnp.concatenate([e[0].astype(jnp.int32), pad])
        dst = jnp.concatenate([e[1].astype(jnp.int32), pad])
        return jnp.stack([src, dst]).reshape(2, NROWS_E, CHUNK)

    idx_all = jnp.stack([prep_idx(edge_index_i2u), prep_idx(edge_index_u2i)])

    degs = _sc_degrees(idx_all).reshape(4, NP)
    degs_src = degs[jnp.array([0, 2])][:, :, None]
    degs_dst = degs[jnp.array([1, 3])][:, :, None]

    feats = jnp.stack([
        jnp.pad(item_feat, ((0, NP - N), (0, 0))),
        jnp.pad(user_feat, ((0, NP - N), (0, 0))),
    ])
    h = _tc_prep(feats, degs_src)
    agg = _sc_aggregate(h, idx_all)

    W = jnp.stack([W_i2u, W_u2i])
    b = jnp.stack([b_i2u, b_u2i])[:, None, :]
    out = _tc_final(agg, degs_dst, W, b)
    return out[0], out[1]


# speedup vs baseline: 11.1820x; 1.2451x over previous
import functools

import jax
import jax.numpy as jnp
from jax import lax
from jax.experimental import pallas as pl
from jax.experimental.pallas import tpu as pltpu
from jax.experimental.pallas import tpu_sc as plsc

N = 10000
D = 128
E = 320000
NTILES = 16
NP = 10240
ROWS_PT = NP // NTILES
CHUNK = 128
CPT = 160
GRP = 40
EP = NTILES * CPT * CHUNK
NROWS_E = EP // CHUNK

_sc_mesh = plsc.VectorSubcoreMesh(core_axis_name="c", subcore_axis_name="s")


@functools.partial(
    pl.kernel,
    out_type=jax.ShapeDtypeStruct((4, 1, NP), jnp.float32),
    mesh=_sc_mesh,
    scratch_types=[
        pltpu.VMEM_SHARED((NP,), jnp.float32),
        pltpu.VMEM_SHARED((NP,), jnp.float32),
        pltpu.VMEM((CPT, CHUNK), jnp.int32),
        pltpu.VMEM((CHUNK,), jnp.float32),
        pltpu.VMEM((640,), jnp.float32),
    ],
)
def _sc_degrees(idx_hbm, deg_hbm, hist_src, hist_dst, idx_v, ones_v, zbuf_v):
    c = lax.axis_index("c")
    s = lax.axis_index("s")
    row0 = s * ROWS_PT

    for j in range(CHUNK // 16):
        ones_v[pl.ds(j * 16, 16)] = jnp.ones((16,), jnp.float32)

    def fill_zeros(i, carry):
        zbuf_v[pl.ds(i * 16, 16)] = jnp.zeros((16,), jnp.float32)
        return carry

    lax.fori_loop(0, 640 // 16, fill_zeros, 0)
    pltpu.sync_copy(zbuf_v.at[pl.ds(0, ROWS_PT)], hist_src.at[pl.ds(row0, ROWS_PT)])
    pltpu.sync_copy(zbuf_v.at[pl.ds(0, ROWS_PT)], hist_dst.at[pl.ds(row0, ROWS_PT)])
    plsc.subcore_barrier()

    pltpu.sync_copy(idx_hbm.at[c, 0, pl.ds(s * CPT, CPT)], idx_v)

    def scat_src(ch, carry):
        pltpu.sync_copy(ones_v, hist_src.at[idx_v.at[ch]], add=True)
        return carry

    lax.fori_loop(0, CPT, scat_src, 0)

    pltpu.sync_copy(idx_hbm.at[c, 1, pl.ds(s * CPT, CPT)], idx_v)

    def scat_dst(ch, carry):
        pltpu.sync_copy(ones_v, hist_dst.at[idx_v.at[ch]], add=True)
        return carry

    lax.fori_loop(0, CPT, scat_dst, 0)
    plsc.subcore_barrier()

    pltpu.sync_copy(hist_src.at[pl.ds(row0, ROWS_PT)],
                    deg_hbm.at[2 * c, 0, pl.ds(row0, ROWS_PT)])
    pltpu.sync_copy(hist_dst.at[pl.ds(row0, ROWS_PT)],
                    deg_hbm.at[2 * c + 1, 0, pl.ds(row0, ROWS_PT)])


@functools.partial(
    pl.kernel,
    out_type=jax.ShapeDtypeStruct((2, NP, D), jnp.float32),
    mesh=_sc_mesh,
    scratch_types=[
        pltpu.VMEM_SHARED((NP, D), jnp.float32),
        pltpu.VMEM((GRP, CHUNK), jnp.int32),
        pltpu.VMEM((GRP, CHUNK), jnp.int32),
        pltpu.VMEM((CHUNK, D), jnp.float32),
        pltpu.VMEM((CHUNK, D), jnp.float32),
        pltpu.SemaphoreType.DMA,
        pltpu.SemaphoreType.DMA,
    ],
)
def _sc_aggregate(h_hbm, idx_hbm, agg_hbm, agg_sh, sidx_v, didx_v,
                  stage0, stage1, semg0, semg1):
    c = lax.axis_index("c")
    s = lax.axis_index("s")
    row0 = s * ROWS_PT

    def zero_row(i, carry):
        for j in range(D // 16):
            stage0[i, pl.ds(j * 16, 16)] = jnp.zeros((16,), jnp.float32)
        return carry

    lax.fori_loop(0, CHUNK, zero_row, 0)
    for k in range(ROWS_PT // CHUNK):
        pltpu.sync_copy(stage0, agg_sh.at[pl.ds(row0 + k * CHUNK, CHUNK)])
    plsc.subcore_barrier()

    for g in range(CPT // GRP):
        base = s * CPT + g * GRP
        pltpu.sync_copy(idx_hbm.at[c, 0, pl.ds(base, GRP)], sidx_v)
        pltpu.sync_copy(idx_hbm.at[c, 1, pl.ds(base, GRP)], didx_v)
        pltpu.async_copy(h_hbm.at[c].at[sidx_v.at[0]], stage0, semg0)

        def pair(p, carry):
            ch0 = 2 * p
            ch1 = 2 * p + 1
            pltpu.make_async_copy(h_hbm.at[c].at[sidx_v.at[ch0]], stage0, semg0).wait()
            pltpu.async_copy(h_hbm.at[c].at[sidx_v.at[ch1]], stage1, semg1)
            pltpu.sync_copy(stage0, agg_sh.at[didx_v.at[ch0]], add=True)
            pltpu.make_async_copy(h_hbm.at[c].at[sidx_v.at[ch1]], stage1, semg1).wait()

            @pl.when(p < GRP // 2 - 1)
            def _prefetch_even():
                pltpu.async_copy(h_hbm.at[c].at[sidx_v.at[ch0 + 2]], stage0, semg0)

            pltpu.sync_copy(stage1, agg_sh.at[didx_v.at[ch1]], add=True)
            return carry

        lax.fori_loop(0, GRP // 2, pair, 0)

    plsc.subcore_barrier()
    pltpu.sync_copy(agg_sh.at[pl.ds(row0, ROWS_PT)],
                    agg_hbm.at[c, pl.ds(row0, ROWS_PT)])


_PBLK = NP // 8


def _tc_prep_body(feat_ref, deg_ref, h_ref):
    d = deg_ref[0]
    h_ref[0] = feat_ref[0] * lax.rsqrt(jnp.maximum(d, 1.0))


def _tc_prep(feats, degs_src):
    return pl.pallas_call(
        _tc_prep_body,
        grid=(2, NP // _PBLK),
        in_specs=[
            pl.BlockSpec((1, _PBLK, D), lambda r, i: (r, i, 0)),
            pl.BlockSpec((1, _PBLK, 1), lambda r, i: (r, i, 0)),
        ],
        out_specs=pl.BlockSpec((1, _PBLK, D), lambda r, i: (r, i, 0)),
        out_shape=jax.ShapeDtypeStruct((2, NP, D), jnp.float32),
    )(feats, degs_src)


_FBLK = 1000


def _tc_final_body(agg_ref, deg_ref, w_ref, b_ref, out_ref):
    a = agg_ref[0] * lax.rsqrt(jnp.maximum(deg_ref[0], 1.0))
    out_ref[0] = jnp.dot(a, w_ref[0], preferred_element_type=jnp.float32) + b_ref[0]


def _tc_final(agg, degs_dst, W, b):
    return pl.pallas_call(
        _tc_final_body,
        grid=(2, N // _FBLK),
        in_specs=[
            pl.BlockSpec((1, _FBLK, D), lambda r, i: (r, i, 0)),
            pl.BlockSpec((1, _FBLK, 1), lambda r, i: (r, i, 0)),
            pl.BlockSpec((1, D, D), lambda r, i: (r, 0, 0)),
            pl.BlockSpec((1, 1, D), lambda r, i: (r, 0, 0)),
        ],
        out_specs=pl.BlockSpec((1, _FBLK, D), lambda r, i: (r, i, 0)),
        out_shape=jax.ShapeDtypeStruct((2, N, D), jnp.float32),
    )(agg, degs_dst, W, b)


def kernel(item_feat, user_feat, edge_index_i2u, edge_index_u2i,
           W_i2u, b_i2u, W_u2i, b_u2i):
    pad = (jnp.arange(EP - E, dtype=jnp.int32) % (NP - N)) + N

    def prep_idx(e):
        src = jnp.concatenate([e[0].astype(jnp.int32), pad])
        dst = jnp.concatenate([e[1].astype(jnp.int32), pad])
        return jnp.stack([src, dst]).reshape(2, NROWS_E, CHUNK)

    idx_all = jnp.stack([prep_idx(edge_index_i2u), prep_idx(edge_index_u2i)])

    degs = _sc_degrees(idx_all).reshape(4, NP)
    degs_src = degs[jnp.array([0, 2])][:, :, None]
    degs_dst = degs[jnp.array([1, 3])][:, :, None]

    feats = jnp.stack([
        jnp.pad(item_feat, ((0, NP - N), (0, 0))),
        jnp.pad(user_feat, ((0, NP - N), (0, 0))),
    ])
    h = _tc_prep(feats, degs_src)
    agg = _sc_aggregate(h, idx_all)

    W = jnp.stack([W_i2u, W_u2i])
    b = jnp.stack([b_i2u, b_u2i])[:, None, :]
    out = _tc_final(agg, degs_dst, W, b)
    return out[0], out[1]
